# baseline (speedup 1.0000x reference)
import functools

import jax
import jax.numpy as jnp
from jax import lax
from jax.experimental import pallas as pl
from jax.experimental.pallas import tpu as pltpu
from jax.experimental.pallas import tpu_sc as plsc

N = 10000
E = 320000
D = 128
R = 8
NB = 4
G = 512

NC = 2
NS = 16
NW = NC * NS
L = 16

EC = 80
EW = E // NW
ECHUNKS = EW // EC
NCHUNKS = N // EC

CNTP = 81920

EC1 = 400
ECHUNKS1 = EW // EC1

_mesh = plsc.VectorSubcoreMesh(core_axis_name="c", subcore_axis_name="s")
_params = pltpu.CompilerParams(needs_layout_passes=False)


def _zero_fill(ref, nvec):
    zero16 = jnp.zeros((L,), jnp.float32)

    def body(i, carry):
        for u in range(4):
            ref[pl.ds((i * 4 + u) * L, L)] = zero16
        return carry

    lax.fori_loop(0, nvec // 4, body, 0)
    for u in range(nvec - (nvec // 4) * 4):
        ref[pl.ds(((nvec // 4) * 4 + u) * L, L)] = zero16


@functools.partial(
    pl.kernel,
    out_type=[
        jax.ShapeDtypeStruct((NC, CNTP), jnp.float32),
        jax.ShapeDtypeStruct((NC, G, D), jnp.float32),
        jax.ShapeDtypeStruct((NC, G), jnp.float32),
    ],
    mesh=_mesh,
    compiler_params=_params,
    scratch_types=[
        pltpu.VMEM_SHARED((CNTP,), jnp.float32),
        pltpu.VMEM_SHARED((G, D), jnp.float32),
        pltpu.VMEM_SHARED((G,), jnp.float32),
        pltpu.VMEM((EC1,), jnp.int32),
        pltpu.VMEM((EC1,), jnp.int32),
        pltpu.VMEM((EC1,), jnp.int32),
        pltpu.VMEM((EC1,), jnp.int32),
        pltpu.VMEM((EC1,), jnp.int32),
        pltpu.VMEM((EC1,), jnp.int32),
        pltpu.VMEM((EC1 // EC, EC), jnp.int32),
        pltpu.VMEM((EC1 // EC, EC), jnp.int32),
        pltpu.VMEM((EC1 // EC, EC), jnp.int32),
        pltpu.VMEM((EC,), jnp.int32),
        pltpu.VMEM((EC,), jnp.float32),
        pltpu.VMEM((EC, D), jnp.float32),
        pltpu.VMEM((CNTP // NS,), jnp.float32),
        pltpu.VMEM((G // NS, D), jnp.float32),
        pltpu.VMEM((G // NS,), jnp.float32),
        pltpu.SemaphoreType.DMA,
        pltpu.SemaphoreType.DMA,
        pltpu.SemaphoreType.DMA,
        pltpu.SemaphoreType.DMA,
        pltpu.SemaphoreType.DMA,
        pltpu.SemaphoreType.DMA,
    ],
)
def _k1(dst_h, typ_h, batch_h, x_h, cnt_o, pool_o, ncnt_o,
        cnt_s, pool_s, ncnt_s,
        dst0, dst1, dst2, typ0, typ1, typ2, hidx0, hidx1, hidx2,
        gbuf, ones_v, xrow_v, bc_v, bp_v, bn_v,
        si0, si1, si2, ss0, ss1, ss2):
    c = lax.axis_index("c")
    s = lax.axis_index("s")
    wid = s * NC + c
    dstb = [dst0, dst1, dst2]
    typb = [typ0, typ1, typ2]
    hidxb = [hidx0, hidx1, hidx2]
    sem_i = [si0, si1, si2]
    sem_s = [ss0, ss1, ss2]

    _zero_fill(bc_v, (CNTP // NS) // L)
    _zero_fill(bn_v, (G // NS) // L)
    one16 = jnp.full((L,), 1.0, jnp.float32)
    for u in range(EC // L):
        ones_v[pl.ds(u * L, L)] = one16

    def zp_body(i, carry):
        for k in range(D // L):
            bp_v[i, pl.ds(k * L, L)] = jnp.zeros((L,), jnp.float32)
        return carry

    lax.fori_loop(0, G // NS, zp_body, 0)

    pltpu.sync_copy(bc_v, cnt_s.at[pl.ds(s * (CNTP // NS), CNTP // NS)])
    pltpu.sync_copy(bp_v, pool_s.at[pl.ds(s * (G // NS), G // NS)])
    pltpu.sync_copy(bn_v, ncnt_s.at[pl.ds(s * (G // NS), G // NS)])
    plsc.subcore_barrier()

    NSUB = EC1 // EC

    def eloads(ci, bb):
        base = wid * EW + ci * EC1
        pltpu.async_copy(dst_h.at[pl.ds(base, EC1)], dstb[bb], sem_i[bb])
        pltpu.async_copy(typ_h.at[pl.ds(base, EC1)], typb[bb], sem_i[bb])

    def eloads_wait(bb):
        pltpu.make_async_copy(dst_h.at[pl.ds(0, EC1)], dstb[bb], sem_i[bb]).wait()
        pltpu.make_async_copy(typ_h.at[pl.ds(0, EC1)], typb[bb], sem_i[bb]).wait()

    def slot(ci, bb, bb1, bb2):
        @pl.when((ci >= 2) & (ci <= ECHUNKS1 + 1))
        def _():
            for j in range(NSUB):
                pltpu.make_async_copy(ones_v, cnt_s.at[hidxb[bb1].at[j]],
                                      sem_s[bb1]).wait()

        @pl.when(ci + 1 <= ECHUNKS1 - 1)
        def _():
            eloads(ci + 1, bb1)

        @pl.when(ci <= ECHUNKS1 - 1)
        def _():
            eloads_wait(bb)
            for j in range(NSUB):
                for k in range(EC // L):
                    d16 = dstb[bb][pl.ds((j * EC // L + k) * L, L)]
                    t16 = typb[bb][pl.ds((j * EC // L + k) * L, L)]
                    hidxb[bb][j, pl.ds(k * L, L)] = d16 * R + t16
            for j in range(NSUB):
                pltpu.async_copy(ones_v, cnt_s.at[hidxb[bb].at[j]],
                                 sem_s[bb], add=True)

    eloads(0, 0)

    def loop(c3, carry):
        for b in range(3):
            ci = c3 * 3 + b
            slot(ci, b, (b + 1) % 3, (b + 2) % 3)
        return carry

    lax.fori_loop(0, (ECHUNKS1 + 2 + 2) // 3 + 1, loop, 0)

    def nchunk(j, carry):
        ci = wid + NW * j

        @pl.when(ci < NCHUNKS)
        def _():
            nb = ci * EC
            pltpu.sync_copy(batch_h.at[pl.ds(nb, EC)], gbuf)
            pltpu.sync_copy(x_h.at[pl.ds(nb, EC)], xrow_v)
            pltpu.sync_copy(xrow_v, pool_s.at[gbuf], add=True)
            pltpu.sync_copy(ones_v, ncnt_s.at[gbuf], add=True)

        return carry

    lax.fori_loop(0, (NCHUNKS + NW - 1) // NW, nchunk, 0)
    plsc.subcore_barrier()

    pltpu.sync_copy(cnt_s.at[pl.ds(s * (CNTP // NS), CNTP // NS)], bc_v)
    pltpu.sync_copy(bc_v, cnt_o.at[c, pl.ds(s * (CNTP // NS), CNTP // NS)])
    pltpu.sync_copy(pool_s.at[pl.ds(s * (G // NS), G // NS)], bp_v)
    pltpu.sync_copy(bp_v, pool_o.at[c, pl.ds(s * (G // NS), G // NS)])
    pltpu.sync_copy(ncnt_s.at[pl.ds(s * (G // NS), G // NS)], bn_v)
    pltpu.sync_copy(bn_v, ncnt_o.at[c, pl.ds(s * (G // NS), G // NS)])


def _k2_body(cnt_ref, pool_ref, ncnt_ref, root_ref, bias_ref, winv_ref, s_ref):
    ctot = cnt_ref[0] + cnt_ref[1]
    winv_ref[...] = 1.0 / jnp.maximum(ctot, 1.0)
    p = pool_ref[0] + pool_ref[1]
    nc = ncnt_ref[0] + ncnt_ref[1]
    s_ref[...] = (jnp.dot(p, root_ref[...], preferred_element_type=jnp.float32)
                  + nc * bias_ref[...])


@functools.partial(
    pl.kernel,
    out_type=[
        jax.ShapeDtypeStruct((E,), jnp.int32),
        jax.ShapeDtypeStruct((E,), jnp.float32),
    ],
    mesh=_mesh,
    compiler_params=_params,
    scratch_types=[
        pltpu.VMEM((N,), jnp.int32),
        pltpu.VMEM((CNTP,), jnp.float32),
        pltpu.VMEM((EC1,), jnp.int32),
        pltpu.VMEM((EC1,), jnp.int32),
        pltpu.VMEM((EC1,), jnp.int32),
        pltpu.VMEM((EC1,), jnp.int32),
        pltpu.VMEM((EC1,), jnp.int32),
        pltpu.VMEM((EC1,), jnp.int32),
        pltpu.VMEM((EC1,), jnp.int32),
        pltpu.VMEM((EC1,), jnp.int32),
        pltpu.VMEM((EC1,), jnp.int32),
        pltpu.VMEM((EC1,), jnp.float32),
        pltpu.VMEM((EC1,), jnp.float32),
        pltpu.VMEM((EC1,), jnp.float32),
        pltpu.SemaphoreType.DMA,
        pltpu.SemaphoreType.DMA,
        pltpu.SemaphoreType.DMA,
        pltpu.SemaphoreType.DMA,
        pltpu.SemaphoreType.DMA,
        pltpu.SemaphoreType.DMA,
    ],
)
def _k2b(dst_h, typ_h, batch_h, winv_h, pidx_o, w_o,
         batch_v, winv_v,
         dst0, dst1, dst2, typ0, typ1, typ2, pidx0, pidx1, pidx2,
         w0, w1, w2, si0, si1, si2, so0, so1, so2):
    c = lax.axis_index("c")
    s = lax.axis_index("s")
    wid = s * NC + c
    dstb = [dst0, dst1, dst2]
    typb = [typ0, typ1, typ2]
    pidxb = [pidx0, pidx1, pidx2]
    wb = [w0, w1, w2]
    sem_i = [si0, si1, si2]
    sem_o = [so0, so1, so2]

    pltpu.sync_copy(batch_h, batch_v)
    pltpu.sync_copy(winv_h, winv_v)

    def eloads(ci, bb):
        base = wid * EW + ci * EC1
        pltpu.async_copy(dst_h.at[pl.ds(base, EC1)], dstb[bb], sem_i[bb])
        pltpu.async_copy(typ_h.at[pl.ds(base, EC1)], typb[bb], sem_i[bb])

    def eloads_wait(bb):
        pltpu.make_async_copy(dst_h.at[pl.ds(0, EC1)], dstb[bb], sem_i[bb]).wait()
        pltpu.make_async_copy(typ_h.at[pl.ds(0, EC1)], typb[bb], sem_i[bb]).wait()

    def slot(ci, bb, bb1, bb2):
        @pl.when((ci >= 2) & (ci <= ECHUNKS1 + 1))
        def _():
            pltpu.make_async_copy(pidxb[bb1],
                                  pidx_o.at[pl.ds(0, EC1)],
                                  sem_o[bb1]).wait()
            pltpu.make_async_copy(wb[bb1],
                                  w_o.at[pl.ds(0, EC1)],
                                  sem_o[bb1]).wait()

        @pl.when(ci + 1 <= ECHUNKS1 - 1)
        def _():
            eloads(ci + 1, bb1)

        @pl.when(ci <= ECHUNKS1 - 1)
        def _():
            eloads_wait(bb)

            def grp(k, carry):
                for u in range(5):
                    off = (k * 5 + u) * L
                    d16 = dstb[bb][pl.ds(off, L)]
                    t16 = typb[bb][pl.ds(off, L)]
                    g16 = plsc.load_gather(batch_v, [d16])
                    pidxb[bb][pl.ds(off, L)] = g16 * R + t16
                    wb[bb][pl.ds(off, L)] = plsc.load_gather(
                        winv_v, [d16 * R + t16])
                return carry

            lax.fori_loop(0, EC1 // (5 * L), grp, 0)
            base = wid * EW + ci * EC1
            pltpu.async_copy(pidxb[bb], pidx_o.at[pl.ds(base, EC1)],
                             sem_o[bb])
            pltpu.async_copy(wb[bb], w_o.at[pl.ds(base, EC1)], sem_o[bb])

    eloads(0, 0)

    def loop(c3, carry):
        for b in range(3):
            ci = c3 * 3 + b
            slot(ci, b, (b + 1) % 3, (b + 2) % 3)
        return carry

    lax.fori_loop(0, (ECHUNKS1 + 2 + 2) // 3 + 1, loop, 0)


@functools.partial(
    pl.kernel,
    out_type=jax.ShapeDtypeStruct((NC, 2 * G * R, D), jnp.float32),
    mesh=_mesh,
    compiler_params=_params,
    scratch_types=[
        pltpu.VMEM_SHARED((2 * G * R, D), jnp.float32),
        pltpu.VMEM((EC,), jnp.int32),
        pltpu.VMEM((EC,), jnp.int32),
        pltpu.VMEM((EC,), jnp.int32),
        pltpu.VMEM((EC,), jnp.int32),
        pltpu.VMEM((EC,), jnp.int32),
        pltpu.VMEM((EC,), jnp.int32),
        pltpu.VMEM((EC,), jnp.int32),
        pltpu.VMEM((EC,), jnp.int32),
        pltpu.VMEM((EC,), jnp.float32),
        pltpu.VMEM((EC,), jnp.float32),
        pltpu.VMEM((EC,), jnp.float32),
        pltpu.VMEM((EC,), jnp.float32),
        pltpu.VMEM((EC, D), jnp.float32),
        pltpu.VMEM((EC, D), jnp.float32),
        pltpu.VMEM((EC, D), jnp.float32),
        pltpu.VMEM((EC, D), jnp.float32),
        pltpu.VMEM((G * R // (2 * NS), D), jnp.float32),
        pltpu.SemaphoreType.DMA,
        pltpu.SemaphoreType.DMA,
        pltpu.SemaphoreType.DMA,
        pltpu.SemaphoreType.DMA,
        pltpu.SemaphoreType.DMA,
        pltpu.SemaphoreType.DMA,
        pltpu.SemaphoreType.DMA,
        pltpu.SemaphoreType.DMA,
        pltpu.SemaphoreType.DMA,
        pltpu.SemaphoreType.DMA,
        pltpu.SemaphoreType.DMA,
        pltpu.SemaphoreType.DMA,
    ],
)
def _k3(src_h, pidx_h, w_h, x_h, p_o,
        p_s,
        src0, src1, src2, src3, pidx0, pidx1, pidx2, pidx3,
        w0, w1, w2, w3, rows0, rows1, rows2, rows3, zb_v,
        si0, si1, si2, si3, sg0, sg1, sg2, sg3, ss0, ss1, ss2, ss3):
    c = lax.axis_index("c")
    s = lax.axis_index("s")
    wid = s * NC + c
    srcb = [src0, src1, src2, src3]
    pidxb = [pidx0, pidx1, pidx2, pidx3]
    wbuf = [w0, w1, w2, w3]
    rowsb = [rows0, rows1, rows2, rows3]
    sem_i = [si0, si1, si2, si3]
    sem_g = [sg0, sg1, sg2, sg3]
    sem_s = [ss0, ss1, ss2, ss3]
    prows = G * R // (2 * NS)

    def zb_body(i, carry):
        for k in range(D // L):
            zb_v[i, pl.ds(k * L, L)] = jnp.zeros((L,), jnp.float32)
        return carry

    lax.fori_loop(0, prows, zb_body, 0)
    for q in range(4):
        pltpu.sync_copy(zb_v, p_s.at[pl.ds(s * 4 * prows + q * prows, prows)])
    plsc.subcore_barrier()
    poff = (s % 2) * (G * R)

    def iloads(ci, bb):
        base = wid * EW + ci * EC
        pltpu.async_copy(src_h.at[pl.ds(base, EC)], srcb[bb], sem_i[bb])
        pltpu.async_copy(pidx_h.at[pl.ds(base, EC)], pidxb[bb], sem_i[bb])
        pltpu.async_copy(w_h.at[pl.ds(base, EC)], wbuf[bb], sem_i[bb])

    def iloads_wait(bb):
        pltpu.make_async_copy(src_h.at[pl.ds(0, EC)], srcb[bb], sem_i[bb]).wait()
        pltpu.make_async_copy(pidx_h.at[pl.ds(0, EC)], pidxb[bb], sem_i[bb]).wait()
        pltpu.make_async_copy(w_h.at[pl.ds(0, EC)], wbuf[bb], sem_i[bb]).wait()

    def slot(ci, b, b1, b2):
        @pl.when((ci >= 2) & (ci <= ECHUNKS + 1))
        def _():
            pltpu.make_async_copy(rowsb[b2], p_s.at[pidxb[b2]],
                                  sem_s[b2]).wait()

        @pl.when(ci + 2 <= ECHUNKS - 1)
        def _():
            iloads(ci + 2, b2)

        @pl.when(ci <= ECHUNKS - 1)
        def _():
            for k in range(EC // L):
                pidxb[b][pl.ds(k * L, L)] = pidxb[b][pl.ds(k * L, L)] + poff
            pltpu.make_async_copy(x_h.at[srcb[b]], rowsb[b], sem_g[b]).wait()

            def scale(jj, cc):
                for u in range(8):
                    j = jj * 8 + u
                    wj = plsc.load_gather(
                        wbuf[b], [jnp.full((L,), j, jnp.int32)])
                    for k in range(D // L):
                        rowsb[b][j, pl.ds(k * L, L)] = (
                            rowsb[b][j, pl.ds(k * L, L)] * wj)
                return cc

            lax.fori_loop(0, EC // 8, scale, 0)
            pltpu.async_copy(rowsb[b], p_s.at[pidxb[b]], sem_s[b], add=True)

        @pl.when(ci + 2 <= ECHUNKS - 1)
        def _():
            iloads_wait(b2)
            pltpu.async_copy(x_h.at[srcb[b2]], rowsb[b2], sem_g[b2])

    iloads(0, 0)
    iloads(1, 1)
    iloads_wait(0)
    pltpu.async_copy(x_h.at[srcb[0]], rowsb[0], sem_g[0])
    iloads_wait(1)
    pltpu.async_copy(x_h.at[srcb[1]], rowsb[1], sem_g[1])

    def loop(c4, carry):
        for b in range(4):
            ci = c4 * 4 + b
            slot(ci, b, (b + 1) % 4, (b + 2) % 4)
        return carry

    lax.fori_loop(0, (ECHUNKS + 2 + 3) // 4 + 1, loop, 0)
    plsc.subcore_barrier()

    for q in range(4):
        off = s * 4 * prows + q * prows
        pltpu.sync_copy(p_s.at[pl.ds(off, prows)], zb_v)
        pltpu.sync_copy(zb_v, p_o.at[c, pl.ds(off, prows)])


def _k4_body(p_ref, w_ref, s_ref, out_ref):
    p = (p_ref[0] + p_ref[1]) + (p_ref[2] + p_ref[3])
    out_ref[...] = s_ref[...] + jnp.dot(p, w_ref[...],
                                        preferred_element_type=jnp.float32)


def kernel(query_node_embeddings, edge_index, edge_type, batch_ids,
           comp, bases, root, bias):
    x = query_node_embeddings.astype(jnp.float32)
    src = edge_index[0].astype(jnp.int32)
    dst = edge_index[1].astype(jnp.int32)
    typ = edge_type.astype(jnp.int32)
    b = batch_ids.astype(jnp.int32)

    cnt_p, pool_p, ncnt_p = _k1(dst, typ, b, x)

    winv640, s_mat = pl.pallas_call(
        _k2_body,
        out_shape=[
            jax.ShapeDtypeStruct((CNTP // D, D), jnp.float32),
            jax.ShapeDtypeStruct((G, D), jnp.float32),
        ],
    )(cnt_p.reshape(NC, CNTP // D, D), pool_p,
      ncnt_p.reshape(NC, G, 1), root, bias.reshape(1, D))

    pidx_all, w_all = _k2b(dst, typ, b, winv640.reshape(CNTP))
    p_part = _k3(src, pidx_all, w_all, x)

    wstack = (comp @ bases.reshape(NB, -1)).reshape(R * D, D)
    pooled = pl.pallas_call(
        _k4_body,
        out_shape=jax.ShapeDtypeStruct((G, D), jnp.float32),
    )(p_part.reshape(2 * NC, G, R * D), wstack, s_mat)
    return pooled


# speedup vs baseline: 1.0420x; 1.0420x over previous
import functools

import jax
import jax.numpy as jnp
from jax import lax
from jax.experimental import pallas as pl
from jax.experimental.pallas import tpu as pltpu
from jax.experimental.pallas import tpu_sc as plsc

N = 10000
E = 320000
D = 128
R = 8
NB = 4
G = 512

NC = 2
NS = 16
NW = NC * NS
L = 16

EC = 80
EW = E // NW
ECHUNKS = EW // EC
NCHUNKS = N // EC

CNTP = 81920

EC1 = 400
ECHUNKS1 = EW // EC1

_mesh = plsc.VectorSubcoreMesh(core_axis_name="c", subcore_axis_name="s")
_params = pltpu.CompilerParams(needs_layout_passes=False)


def _zero_fill(ref, nvec):
    zero16 = jnp.zeros((L,), jnp.float32)

    def body(i, carry):
        for u in range(4):
            ref[pl.ds((i * 4 + u) * L, L)] = zero16
        return carry

    lax.fori_loop(0, nvec // 4, body, 0)
    for u in range(nvec - (nvec // 4) * 4):
        ref[pl.ds(((nvec // 4) * 4 + u) * L, L)] = zero16


@functools.partial(
    pl.kernel,
    out_type=[
        jax.ShapeDtypeStruct((NC, CNTP), jnp.float32),
        jax.ShapeDtypeStruct((NC, G, D), jnp.float32),
        jax.ShapeDtypeStruct((NC, G), jnp.float32),
    ],
    mesh=_mesh,
    compiler_params=_params,
    scratch_types=[
        pltpu.VMEM_SHARED((CNTP,), jnp.float32),
        pltpu.VMEM_SHARED((G, D), jnp.float32),
        pltpu.VMEM_SHARED((G,), jnp.float32),
        pltpu.VMEM((EC1,), jnp.int32),
        pltpu.VMEM((EC1,), jnp.int32),
        pltpu.VMEM((EC1,), jnp.int32),
        pltpu.VMEM((EC1,), jnp.int32),
        pltpu.VMEM((EC1,), jnp.int32),
        pltpu.VMEM((EC1,), jnp.int32),
        pltpu.VMEM((EC1 // EC, EC), jnp.int32),
        pltpu.VMEM((EC1 // EC, EC), jnp.int32),
        pltpu.VMEM((EC1 // EC, EC), jnp.int32),
        pltpu.VMEM((EC,), jnp.int32),
        pltpu.VMEM((EC,), jnp.float32),
        pltpu.VMEM((EC, D), jnp.float32),
        pltpu.VMEM((CNTP // NS,), jnp.float32),
        pltpu.VMEM((G // NS, D), jnp.float32),
        pltpu.VMEM((G // NS,), jnp.float32),
        pltpu.SemaphoreType.DMA,
        pltpu.SemaphoreType.DMA,
        pltpu.SemaphoreType.DMA,
        pltpu.SemaphoreType.DMA,
        pltpu.SemaphoreType.DMA,
        pltpu.SemaphoreType.DMA,
    ],
)
def _k1(dst_h, typ_h, batch_h, x_h, cnt_o, pool_o, ncnt_o,
        cnt_s, pool_s, ncnt_s,
        dst0, dst1, dst2, typ0, typ1, typ2, hidx0, hidx1, hidx2,
        gbuf, ones_v, xrow_v, bc_v, bp_v, bn_v,
        si0, si1, si2, ss0, ss1, ss2):
    c = lax.axis_index("c")
    s = lax.axis_index("s")
    wid = s * NC + c
    dstb = [dst0, dst1, dst2]
    typb = [typ0, typ1, typ2]
    hidxb = [hidx0, hidx1, hidx2]
    sem_i = [si0, si1, si2]
    sem_s = [ss0, ss1, ss2]

    _zero_fill(bc_v, (CNTP // NS) // L)
    _zero_fill(bn_v, (G // NS) // L)
    one16 = jnp.full((L,), 1.0, jnp.float32)
    for u in range(EC // L):
        ones_v[pl.ds(u * L, L)] = one16

    def zp_body(i, carry):
        for k in range(D // L):
            bp_v[i, pl.ds(k * L, L)] = jnp.zeros((L,), jnp.float32)
        return carry

    lax.fori_loop(0, G // NS, zp_body, 0)

    pltpu.sync_copy(bc_v, cnt_s.at[pl.ds(s * (CNTP // NS), CNTP // NS)])
    pltpu.sync_copy(bp_v, pool_s.at[pl.ds(s * (G // NS), G // NS)])
    pltpu.sync_copy(bn_v, ncnt_s.at[pl.ds(s * (G // NS), G // NS)])
    plsc.subcore_barrier()

    NSUB = EC1 // EC

    def eloads(ci, bb):
        base = wid * EW + ci * EC1
        pltpu.async_copy(dst_h.at[pl.ds(base, EC1)], dstb[bb], sem_i[bb])
        pltpu.async_copy(typ_h.at[pl.ds(base, EC1)], typb[bb], sem_i[bb])

    def eloads_wait(bb):
        pltpu.make_async_copy(dst_h.at[pl.ds(0, EC1)], dstb[bb], sem_i[bb]).wait()
        pltpu.make_async_copy(typ_h.at[pl.ds(0, EC1)], typb[bb], sem_i[bb]).wait()

    def slot(ci, bb, bb1, bb2):
        @pl.when((ci >= 2) & (ci <= ECHUNKS1 + 1))
        def _():
            for j in range(NSUB):
                pltpu.make_async_copy(ones_v, cnt_s.at[hidxb[bb1].at[j]],
                                      sem_s[bb1]).wait()

        @pl.when(ci + 1 <= ECHUNKS1 - 1)
        def _():
            eloads(ci + 1, bb1)

        @pl.when(ci <= ECHUNKS1 - 1)
        def _():
            eloads_wait(bb)
            for j in range(NSUB):
                for k in range(EC // L):
                    d16 = dstb[bb][pl.ds((j * EC // L + k) * L, L)]
                    t16 = typb[bb][pl.ds((j * EC // L + k) * L, L)]
                    hidxb[bb][j, pl.ds(k * L, L)] = d16 * R + t16
            for j in range(NSUB):
                pltpu.async_copy(ones_v, cnt_s.at[hidxb[bb].at[j]],
                                 sem_s[bb], add=True)

    eloads(0, 0)

    def loop(c3, carry):
        for b in range(3):
            ci = c3 * 3 + b
            slot(ci, b, (b + 1) % 3, (b + 2) % 3)
        return carry

    lax.fori_loop(0, (ECHUNKS1 + 2 + 2) // 3 + 1, loop, 0)

    def nchunk(j, carry):
        ci = wid + NW * j

        @pl.when(ci < NCHUNKS)
        def _():
            nb = ci * EC
            pltpu.sync_copy(batch_h.at[pl.ds(nb, EC)], gbuf)
            pltpu.sync_copy(x_h.at[pl.ds(nb, EC)], xrow_v)
            pltpu.sync_copy(xrow_v, pool_s.at[gbuf], add=True)
            pltpu.sync_copy(ones_v, ncnt_s.at[gbuf], add=True)

        return carry

    lax.fori_loop(0, (NCHUNKS + NW - 1) // NW, nchunk, 0)
    plsc.subcore_barrier()

    pltpu.sync_copy(cnt_s.at[pl.ds(s * (CNTP // NS), CNTP // NS)], bc_v)
    pltpu.sync_copy(bc_v, cnt_o.at[c, pl.ds(s * (CNTP // NS), CNTP // NS)])
    pltpu.sync_copy(pool_s.at[pl.ds(s * (G // NS), G // NS)], bp_v)
    pltpu.sync_copy(bp_v, pool_o.at[c, pl.ds(s * (G // NS), G // NS)])
    pltpu.sync_copy(ncnt_s.at[pl.ds(s * (G // NS), G // NS)], bn_v)
    pltpu.sync_copy(bn_v, ncnt_o.at[c, pl.ds(s * (G // NS), G // NS)])


def _k2_body(cnt_ref, pool_ref, ncnt_ref, root_ref, bias_ref, winv_ref, s_ref):
    ctot = cnt_ref[0] + cnt_ref[1]
    winv_ref[...] = 1.0 / jnp.maximum(ctot, 1.0)
    p = pool_ref[0] + pool_ref[1]
    nc = ncnt_ref[0] + ncnt_ref[1]
    s_ref[...] = (jnp.dot(p, root_ref[...], preferred_element_type=jnp.float32)
                  + nc * bias_ref[...])


@functools.partial(
    pl.kernel,
    out_type=[
        jax.ShapeDtypeStruct((E,), jnp.int32),
        jax.ShapeDtypeStruct((E,), jnp.float32),
    ],
    mesh=_mesh,
    compiler_params=_params,
    scratch_types=[
        pltpu.VMEM((N,), jnp.int32),
        pltpu.VMEM((CNTP,), jnp.float32),
        pltpu.VMEM((EC1,), jnp.int32),
        pltpu.VMEM((EC1,), jnp.int32),
        pltpu.VMEM((EC1,), jnp.int32),
        pltpu.VMEM((EC1,), jnp.int32),
        pltpu.VMEM((EC1,), jnp.int32),
        pltpu.VMEM((EC1,), jnp.int32),
        pltpu.VMEM((EC1,), jnp.int32),
        pltpu.VMEM((EC1,), jnp.int32),
        pltpu.VMEM((EC1,), jnp.int32),
        pltpu.VMEM((EC1,), jnp.float32),
        pltpu.VMEM((EC1,), jnp.float32),
        pltpu.VMEM((EC1,), jnp.float32),
        pltpu.SemaphoreType.DMA,
        pltpu.SemaphoreType.DMA,
        pltpu.SemaphoreType.DMA,
        pltpu.SemaphoreType.DMA,
        pltpu.SemaphoreType.DMA,
        pltpu.SemaphoreType.DMA,
    ],
)
def _k2b(dst_h, typ_h, batch_h, winv_h, pidx_o, w_o,
         batch_v, winv_v,
         dst0, dst1, dst2, typ0, typ1, typ2, pidx0, pidx1, pidx2,
         w0, w1, w2, si0, si1, si2, so0, so1, so2):
    c = lax.axis_index("c")
    s = lax.axis_index("s")
    wid = s * NC + c
    dstb = [dst0, dst1, dst2]
    typb = [typ0, typ1, typ2]
    pidxb = [pidx0, pidx1, pidx2]
    wb = [w0, w1, w2]
    sem_i = [si0, si1, si2]
    sem_o = [so0, so1, so2]

    pltpu.sync_copy(batch_h, batch_v)
    pltpu.sync_copy(winv_h, winv_v)

    def eloads(ci, bb):
        base = wid * EW + ci * EC1
        pltpu.async_copy(dst_h.at[pl.ds(base, EC1)], dstb[bb], sem_i[bb])
        pltpu.async_copy(typ_h.at[pl.ds(base, EC1)], typb[bb], sem_i[bb])

    def eloads_wait(bb):
        pltpu.make_async_copy(dst_h.at[pl.ds(0, EC1)], dstb[bb], sem_i[bb]).wait()
        pltpu.make_async_copy(typ_h.at[pl.ds(0, EC1)], typb[bb], sem_i[bb]).wait()

    def slot(ci, bb, bb1, bb2):
        @pl.when((ci >= 2) & (ci <= ECHUNKS1 + 1))
        def _():
            pltpu.make_async_copy(pidxb[bb1],
                                  pidx_o.at[pl.ds(0, EC1)],
                                  sem_o[bb1]).wait()
            pltpu.make_async_copy(wb[bb1],
                                  w_o.at[pl.ds(0, EC1)],
                                  sem_o[bb1]).wait()

        @pl.when(ci + 1 <= ECHUNKS1 - 1)
        def _():
            eloads(ci + 1, bb1)

        @pl.when(ci <= ECHUNKS1 - 1)
        def _():
            eloads_wait(bb)

            def grp(k, carry):
                for u in range(5):
                    off = (k * 5 + u) * L
                    d16 = dstb[bb][pl.ds(off, L)]
                    t16 = typb[bb][pl.ds(off, L)]
                    g16 = plsc.load_gather(batch_v, [d16])
                    pidxb[bb][pl.ds(off, L)] = g16 * R + t16
                    wb[bb][pl.ds(off, L)] = plsc.load_gather(
                        winv_v, [d16 * R + t16])
                return carry

            lax.fori_loop(0, EC1 // (5 * L), grp, 0)
            base = wid * EW + ci * EC1
            pltpu.async_copy(pidxb[bb], pidx_o.at[pl.ds(base, EC1)],
                             sem_o[bb])
            pltpu.async_copy(wb[bb], w_o.at[pl.ds(base, EC1)], sem_o[bb])

    eloads(0, 0)

    def loop(c3, carry):
        for b in range(3):
            ci = c3 * 3 + b
            slot(ci, b, (b + 1) % 3, (b + 2) % 3)
        return carry

    lax.fori_loop(0, (ECHUNKS1 + 2 + 2) // 3 + 1, loop, 0)


@functools.partial(
    pl.kernel,
    out_type=jax.ShapeDtypeStruct((NC, G * R, D), jnp.float32),
    mesh=_mesh,
    compiler_params=_params,
    scratch_types=[
        pltpu.VMEM_SHARED((G * R, D), jnp.float32),
        pltpu.VMEM((EC,), jnp.int32),
        pltpu.VMEM((EC,), jnp.int32),
        pltpu.VMEM((EC,), jnp.int32),
        pltpu.VMEM((EC,), jnp.int32),
        pltpu.VMEM((EC,), jnp.int32),
        pltpu.VMEM((EC,), jnp.int32),
        pltpu.VMEM((EC,), jnp.int32),
        pltpu.VMEM((EC,), jnp.int32),
        pltpu.VMEM((EC,), jnp.float32),
        pltpu.VMEM((EC,), jnp.float32),
        pltpu.VMEM((EC,), jnp.float32),
        pltpu.VMEM((EC,), jnp.float32),
        pltpu.VMEM((EC, D), jnp.float32),
        pltpu.VMEM((EC, D), jnp.float32),
        pltpu.VMEM((EC, D), jnp.float32),
        pltpu.VMEM((EC, D), jnp.float32),
        pltpu.VMEM((G * R // (2 * NS), D), jnp.float32),
        pltpu.SemaphoreType.DMA,
        pltpu.SemaphoreType.DMA,
        pltpu.SemaphoreType.DMA,
        pltpu.SemaphoreType.DMA,
        pltpu.SemaphoreType.DMA,
        pltpu.SemaphoreType.DMA,
        pltpu.SemaphoreType.DMA,
        pltpu.SemaphoreType.DMA,
        pltpu.SemaphoreType.DMA,
        pltpu.SemaphoreType.DMA,
        pltpu.SemaphoreType.DMA,
        pltpu.SemaphoreType.DMA,
    ],
)
def _k3(src_h, pidx_h, w_h, x_h, p_o,
        p_s,
        src0, src1, src2, src3, pidx0, pidx1, pidx2, pidx3,
        w0, w1, w2, w3, rows0, rows1, rows2, rows3, zb_v,
        si0, si1, si2, si3, sg0, sg1, sg2, sg3, ss0, ss1, ss2, ss3):
    c = lax.axis_index("c")
    s = lax.axis_index("s")
    wid = s * NC + c
    srcb = [src0, src1, src2, src3]
    pidxb = [pidx0, pidx1, pidx2, pidx3]
    wbuf = [w0, w1, w2, w3]
    rowsb = [rows0, rows1, rows2, rows3]
    sem_i = [si0, si1, si2, si3]
    sem_g = [sg0, sg1, sg2, sg3]
    sem_s = [ss0, ss1, ss2, ss3]
    prows = G * R // (2 * NS)

    def zb_body(i, carry):
        for k in range(D // L):
            zb_v[i, pl.ds(k * L, L)] = jnp.zeros((L,), jnp.float32)
        return carry

    lax.fori_loop(0, prows, zb_body, 0)
    for q in range(2):
        pltpu.sync_copy(zb_v, p_s.at[pl.ds(s * 2 * prows + q * prows, prows)])
    plsc.subcore_barrier()

    def iloads(ci, bb):
        base = wid * EW + ci * EC
        pltpu.async_copy(src_h.at[pl.ds(base, EC)], srcb[bb], sem_i[bb])
        pltpu.async_copy(pidx_h.at[pl.ds(base, EC)], pidxb[bb], sem_i[bb])
        pltpu.async_copy(w_h.at[pl.ds(base, EC)], wbuf[bb], sem_i[bb])

    def iloads_wait(bb):
        pltpu.make_async_copy(src_h.at[pl.ds(0, EC)], srcb[bb], sem_i[bb]).wait()
        pltpu.make_async_copy(pidx_h.at[pl.ds(0, EC)], pidxb[bb], sem_i[bb]).wait()
        pltpu.make_async_copy(w_h.at[pl.ds(0, EC)], wbuf[bb], sem_i[bb]).wait()

    def slot(ci, b, b1, b2):
        @pl.when((ci >= 2) & (ci <= ECHUNKS + 1))
        def _():
            pltpu.make_async_copy(rowsb[b2], p_s.at[pidxb[b2]],
                                  sem_s[b2]).wait()

        @pl.when(ci + 2 <= ECHUNKS - 1)
        def _():
            iloads(ci + 2, b2)

        @pl.when(ci <= ECHUNKS - 1)
        def _():
            pltpu.make_async_copy(x_h.at[srcb[b]], rowsb[b], sem_g[b]).wait()

            def scale(jj, cc):
                for u in range(8):
                    j = jj * 8 + u
                    wj = plsc.load_gather(
                        wbuf[b], [jnp.full((L,), j, jnp.int32)])
                    for k in range(D // L):
                        rowsb[b][j, pl.ds(k * L, L)] = (
                            rowsb[b][j, pl.ds(k * L, L)] * wj)
                return cc

            lax.fori_loop(0, EC // 8, scale, 0)
            pltpu.async_copy(rowsb[b], p_s.at[pidxb[b]], sem_s[b], add=True)

        @pl.when(ci + 2 <= ECHUNKS - 1)
        def _():
            iloads_wait(b2)
            pltpu.async_copy(x_h.at[srcb[b2]], rowsb[b2], sem_g[b2])

    iloads(0, 0)
    iloads(1, 1)
    iloads_wait(0)
    pltpu.async_copy(x_h.at[srcb[0]], rowsb[0], sem_g[0])
    iloads_wait(1)
    pltpu.async_copy(x_h.at[srcb[1]], rowsb[1], sem_g[1])

    def loop(c4, carry):
        for b in range(4):
            ci = c4 * 4 + b
            slot(ci, b, (b + 1) % 4, (b + 2) % 4)
        return carry

    lax.fori_loop(0, (ECHUNKS + 2 + 3) // 4 + 1, loop, 0)
    plsc.subcore_barrier()

    for q in range(2):
        off = s * 2 * prows + q * prows
        pltpu.sync_copy(p_s.at[pl.ds(off, prows)], zb_v)
        pltpu.sync_copy(zb_v, p_o.at[c, pl.ds(off, prows)])


def _k4_body(p_ref, w_ref, s_ref, out_ref):
    p = p_ref[0] + p_ref[1]
    out_ref[...] = s_ref[...] + jnp.dot(p, w_ref[...],
                                        preferred_element_type=jnp.float32)


def kernel(query_node_embeddings, edge_index, edge_type, batch_ids,
           comp, bases, root, bias):
    x = query_node_embeddings.astype(jnp.float32)
    src = edge_index[0].astype(jnp.int32)
    dst = edge_index[1].astype(jnp.int32)
    typ = edge_type.astype(jnp.int32)
    b = batch_ids.astype(jnp.int32)

    cnt_p, pool_p, ncnt_p = _k1(dst, typ, b, x)

    winv640, s_mat = pl.pallas_call(
        _k2_body,
        out_shape=[
            jax.ShapeDtypeStruct((CNTP // D, D), jnp.float32),
            jax.ShapeDtypeStruct((G, D), jnp.float32),
        ],
    )(cnt_p.reshape(NC, CNTP // D, D), pool_p,
      ncnt_p.reshape(NC, G, 1), root, bias.reshape(1, D))

    pidx_all, w_all = _k2b(dst, typ, b, winv640.reshape(CNTP))
    p_part = _k3(src, pidx_all, w_all, x)

    wstack = (comp @ bases.reshape(NB, -1)).reshape(R * D, D)
    pooled = pl.pallas_call(
        _k4_body,
        out_shape=jax.ShapeDtypeStruct((G, D), jnp.float32),
    )(p_part.reshape(NC, G, R * D), wstack, s_mat)
    return pooled
